# Initial kernel scaffold; baseline (speedup 1.0000x reference)
import functools

import jax
import jax.numpy as jnp
from jax import lax
from jax.experimental import pallas as pl
from jax.experimental.pallas import tpu as pltpu
from jax.experimental.pallas import tpu_sc as plsc

N = 10000
D = 128
PD = 144
NC, NS, L = 2, 16, 16
NW = NC * NS
E_RAW = 320000
E_REAL = E_RAW + N
CHUNK = 128
NCHUNK = 81
T_TILE = NCHUNK * CHUNK
E_PAD = T_TILE * NW
RPT = N // NS
BR = 1000


def _pre_body(x_ref, w_ref, asw_ref, adw_ref, hp_ref, as_ref, ad_ref):
    h = jnp.dot(x_ref[...], w_ref[...], preferred_element_type=jnp.float32)
    hp_ref[:, :D] = h
    col = lax.broadcasted_iota(jnp.int32, (BR, PD - D), 1)
    hp_ref[:, D:] = jnp.where(col == 0, 1.0, 0.0).astype(jnp.float32)
    as_ref[...] = (h * asw_ref[...]).sum(axis=1)
    ad_ref[...] = (h * adw_ref[...]).sum(axis=1)


def _post_body(part_ref, bias_ref, o_ref):
    p = part_ref[...]
    srow = p[0] + p[1]
    num = srow[:, :D]
    den = srow[:, D:D + 1]
    o_ref[...] = jnp.tanh(num / (den + 1e-16) + bias_ref[...])


def _sc_body(hp_hbm, src_hbm, dst_hbm, asrc_hbm, adst_hbm, g_hbm, out_hbm,
             asv, adv, gv, src_t, dst_t, ea_v, rows_v, acc, sem):
    c = lax.axis_index("c")
    s = lax.axis_index("s")
    wid = c * NS + s
    pltpu.sync_copy(asrc_hbm, asv)
    pltpu.sync_copy(adst_hbm, adv)
    pltpu.sync_copy(g_hbm, gv)
    pltpu.sync_copy(src_hbm.at[wid], src_t)
    pltpu.sync_copy(dst_hbm.at[wid], dst_t)

    def _zrow(i, carry):
        for j in range(PD // L):
            rows_v[i, pl.ds(j * L, L)] = jnp.zeros((L,), jnp.float32)
        return carry
    lax.fori_loop(0, CHUNK, _zrow, 0)
    for b in range(RPT // 125):
        pltpu.sync_copy(rows_v.at[pl.ds(0, 125), :],
                        acc.at[pl.ds(s * RPT + b * 125, 125), :])
    plsc.subcore_barrier()

    gvec = gv[...]
    lane = lax.iota(jnp.int32, L)

    def _chunk(k, carry):
        base = wid * T_TILE + k * CHUNK
        cp = pltpu.async_copy(hp_hbm.at[src_t.at[k]], rows_v, sem)
        for g8 in range(CHUNK // L):
            sidx = src_t[k, pl.ds(g8 * L, L)]
            didx = dst_t[k, pl.ds(g8 * L, L)]
            al = plsc.load_gather(asv, [sidx]) + plsc.load_gather(adv, [didx])
            al = jnp.where(al > 0, al, 0.2 * al)
            ea = jnp.exp(al - gvec)
            eid = base + g8 * L + lane
            ea = jnp.where(eid < E_REAL, ea, 0.0)
            ea_v[pl.ds(g8 * L, L)] = ea
        cp.wait()

        def _scale(e, cc):
            eb = plsc.load_gather(ea_v, [jnp.zeros((L,), jnp.int32) + e])
            for j in range(PD // L):
                rows_v[e, pl.ds(j * L, L)] = rows_v[e, pl.ds(j * L, L)] * eb
            return cc
        lax.fori_loop(0, CHUNK, _scale, 0)
        pltpu.sync_copy(rows_v, acc.at[dst_t.at[k]], add=True)
        return carry
    lax.fori_loop(0, NCHUNK, _chunk, 0)

    plsc.subcore_barrier()
    for b in range(RPT // 125):
        r0 = s * RPT + b * 125
        pltpu.sync_copy(acc.at[pl.ds(r0, 125), :],
                        out_hbm.at[c, pl.ds(r0, 125), :])


def kernel(x, edge_index, W, att_src, att_dst, bias):
    hp, a_src, a_dst = pl.pallas_call(
        _pre_body,
        grid=(N // BR,),
        in_specs=[
            pl.BlockSpec((BR, D), lambda i: (i, 0)),
            pl.BlockSpec((D, D), lambda i: (0, 0)),
            pl.BlockSpec((1, D), lambda i: (0, 0)),
            pl.BlockSpec((1, D), lambda i: (0, 0)),
        ],
        out_specs=[
            pl.BlockSpec((BR, PD), lambda i: (i, 0)),
            pl.BlockSpec((BR,), lambda i: (i,)),
            pl.BlockSpec((BR,), lambda i: (i,)),
        ],
        out_shape=[
            jax.ShapeDtypeStruct((N, PD), jnp.float32),
            jax.ShapeDtypeStruct((N,), jnp.float32),
            jax.ShapeDtypeStruct((N,), jnp.float32),
        ],
    )(x, W, att_src, att_dst)

    bound = jnp.max(a_src) + jnp.max(a_dst)
    g = jnp.where(bound > 0, bound, 0.2 * bound)
    g16 = jnp.full((L,), g, jnp.float32)

    loop = jnp.arange(N, dtype=jnp.int32)
    padz = jnp.zeros((E_PAD - E_REAL,), jnp.int32)
    srcp = jnp.concatenate([edge_index[0], loop, padz]).reshape(NW, NCHUNK, CHUNK)
    dstp = jnp.concatenate([edge_index[1], loop, padz]).reshape(NW, NCHUNK, CHUNK)

    mesh = plsc.VectorSubcoreMesh(core_axis_name="c", subcore_axis_name="s")
    part = pl.kernel(
        _sc_body,
        mesh=mesh,
        out_type=jax.ShapeDtypeStruct((NC, N, PD), jnp.float32),
        scratch_types=[
            pltpu.VMEM((N,), jnp.float32),
            pltpu.VMEM((N,), jnp.float32),
            pltpu.VMEM((L,), jnp.float32),
            pltpu.VMEM((NCHUNK, CHUNK), jnp.int32),
            pltpu.VMEM((NCHUNK, CHUNK), jnp.int32),
            pltpu.VMEM((CHUNK,), jnp.float32),
            pltpu.VMEM((CHUNK, PD), jnp.float32),
            pltpu.VMEM_SHARED((N, PD), jnp.float32),
            pltpu.SemaphoreType.DMA,
        ],
    )(hp, srcp, dstp, a_src, a_dst, g16)

    out = pl.pallas_call(
        _post_body,
        grid=(N // BR,),
        in_specs=[
            pl.BlockSpec((NC, BR, PD), lambda i: (0, i, 0)),
            pl.BlockSpec((D,), lambda i: (0,)),
        ],
        out_specs=pl.BlockSpec((BR, D), lambda i: (i, 0)),
        out_shape=jax.ShapeDtypeStruct((N, D), jnp.float32),
    )(part, bias)
    return out


# speedup vs baseline: 23.3161x; 23.3161x over previous
import functools

import jax
import jax.numpy as jnp
from jax import lax
from jax.experimental import pallas as pl
from jax.experimental.pallas import tpu as pltpu
from jax.experimental.pallas import tpu_sc as plsc

N = 10000
D = 128
PD = 144
NC, NS, L = 2, 16, 16
NW = NC * NS
E_RAW = 320000
E_REAL = E_RAW + N
CHUNK = 128
NCHUNK = 81
T_TILE = NCHUNK * CHUNK
E_PAD = T_TILE * NW
NP = 10240
RPT = NP // NS
BR = 1000
BRP = 1280


def _pre_body(x_ref, w_ref, asw_ref, adw_ref, hp_ref, as_ref, ad_ref):
    h = jnp.dot(x_ref[...], w_ref[...], preferred_element_type=jnp.float32)
    a_s = (h * asw_ref[...]).sum(axis=1)
    a_d = (h * adw_ref[...]).sum(axis=1)
    hp_ref[:, :D] = h
    col = lax.broadcasted_iota(jnp.int32, (BRP, PD - D), 1)
    hp_ref[:, D:] = jnp.where(col == 0, 1.0,
                              jnp.where(col == 1, a_s[:, None], 0.0))
    i = pl.program_id(0)
    as_ref[pl.ds(pl.multiple_of(i * BRP, 128), BRP)] = a_s
    ad_ref[pl.ds(pl.multiple_of(i * BRP, 128), BRP)] = a_d


def _post_body(part_ref, bias_ref, o_ref):
    p = part_ref[...]
    srow = p[0] + p[1]
    num = srow[:, :D]
    den = srow[:, D:D + 1]
    o_ref[...] = jnp.tanh(num / (den + 1e-16) + bias_ref[...])


def _sc_body(hp_hbm, edges_hbm, adst_hbm, g_hbm, out_hbm,
             adv, gv, ed_v, ea_v, rows_v, acc, sem):
    c = lax.axis_index("c")
    s = lax.axis_index("s")
    wid = c * NS + s
    pltpu.sync_copy(adst_hbm, adv)
    pltpu.sync_copy(g_hbm, gv)

    def _zrow(i, carry):
        for j in range(PD // L):
            rows_v[i, pl.ds(j * L, L)] = jnp.zeros((L,), jnp.float32)
        return carry
    lax.fori_loop(0, CHUNK, _zrow, 0)
    for b in range(RPT // CHUNK):
        pltpu.sync_copy(rows_v,
                        acc.at[pl.ds(s * RPT + b * CHUNK, CHUNK), :])
    plsc.subcore_barrier()

    gvec = gv[...]
    lane = lax.iota(jnp.int32, L)

    def _chunk(k, carry):
        base = wid * T_TILE + k * CHUNK
        pltpu.sync_copy(edges_hbm.at[wid, :, pl.ds(k * CHUNK, CHUNK)], ed_v)
        cp = pltpu.async_copy(hp_hbm.at[ed_v.at[0]], rows_v, sem)
        cp.wait()
        for g8 in range(CHUNK // L):
            ridx = g8 * L + lane
            a_s = plsc.load_gather(rows_v, [ridx,
                                            jnp.full((L,), D + 1, jnp.int32)])
            didx = ed_v[1, pl.ds(g8 * L, L)]
            al = a_s + plsc.load_gather(adv, [didx])
            al = jnp.where(al > 0, al, 0.2 * al)
            ea = jnp.exp(al - gvec)
            eid = base + ridx
            ea = jnp.where(eid < E_REAL, ea, 0.0)
            ea_v[pl.ds(g8 * L, L)] = ea

        def _scale(e, cc):
            eb = plsc.load_gather(ea_v, [jnp.zeros((L,), jnp.int32) + e])
            for j in range(PD // L):
                rows_v[e, pl.ds(j * L, L)] = rows_v[e, pl.ds(j * L, L)] * eb
            return cc
        lax.fori_loop(0, CHUNK, _scale, 0)
        pltpu.sync_copy(rows_v, acc.at[ed_v.at[1]], add=True)
        return carry
    lax.fori_loop(0, NCHUNK, _chunk, 0)

    plsc.subcore_barrier()
    for b in range(RPT // CHUNK):
        r0 = s * RPT + b * CHUNK
        pltpu.sync_copy(acc.at[pl.ds(r0, CHUNK), :],
                        out_hbm.at[c, pl.ds(r0, CHUNK), :])


def kernel(x, edge_index, W, att_src, att_dst, bias):
    xp = jnp.concatenate([x, jnp.zeros((NP - N, D), jnp.float32)], axis=0)
    hp, a_src, a_dst = pl.pallas_call(
        _pre_body,
        grid=(NP // BRP,),
        in_specs=[
            pl.BlockSpec((BRP, D), lambda i: (i, 0)),
            pl.BlockSpec((D, D), lambda i: (0, 0)),
            pl.BlockSpec((1, D), lambda i: (0, 0)),
            pl.BlockSpec((1, D), lambda i: (0, 0)),
        ],
        out_specs=[
            pl.BlockSpec((BRP, PD), lambda i: (i, 0)),
            pl.BlockSpec((NP,), lambda i: (0,)),
            pl.BlockSpec((NP,), lambda i: (0,)),
        ],
        out_shape=[
            jax.ShapeDtypeStruct((NP, PD), jnp.float32),
            jax.ShapeDtypeStruct((NP,), jnp.float32),
            jax.ShapeDtypeStruct((NP,), jnp.float32),
        ],
    )(xp, W, att_src, att_dst)

    bound = jnp.max(a_src[:N]) + jnp.max(a_dst[:N])
    g = jnp.where(bound > 0, bound, 0.2 * bound)
    g16 = jnp.full((L,), g, jnp.float32)

    loop = jnp.arange(N, dtype=jnp.int32)
    padz = jnp.zeros((E_PAD - E_REAL,), jnp.int32)
    srcp = jnp.concatenate([edge_index[0], loop, padz]).reshape(NW, T_TILE)
    dstp = jnp.concatenate([edge_index[1], loop, padz]).reshape(NW, T_TILE)
    edges = jnp.stack([srcp, dstp], axis=1)

    mesh = plsc.VectorSubcoreMesh(core_axis_name="c", subcore_axis_name="s")
    part = pl.kernel(
        _sc_body,
        mesh=mesh,
        compiler_params=pltpu.CompilerParams(
            needs_layout_passes=False, use_tc_tiling_on_sc=False),
        out_type=jax.ShapeDtypeStruct((NC, NP, PD), jnp.float32),
        scratch_types=[
            pltpu.VMEM((NP,), jnp.float32),
            pltpu.VMEM((L,), jnp.float32),
            pltpu.VMEM((2, CHUNK), jnp.int32),
            pltpu.VMEM((CHUNK,), jnp.float32),
            pltpu.VMEM((CHUNK, PD), jnp.float32),
            pltpu.VMEM_SHARED((NP, PD), jnp.float32),
            pltpu.SemaphoreType.DMA,
        ],
    )(hp, edges, a_dst, g16)

    out = pl.pallas_call(
        _post_body,
        grid=(N // BR,),
        in_specs=[
            pl.BlockSpec((NC, BR, PD), lambda i: (0, i, 0)),
            pl.BlockSpec((D,), lambda i: (0,)),
        ],
        out_specs=pl.BlockSpec((BR, D), lambda i: (i, 0)),
        out_shape=jax.ShapeDtypeStruct((N, D), jnp.float32),
    )(part, bias)
    return out


# speedup vs baseline: 28.4628x; 1.2207x over previous
import functools

import jax
import jax.numpy as jnp
from jax import lax
from jax.experimental import pallas as pl
from jax.experimental.pallas import tpu as pltpu
from jax.experimental.pallas import tpu_sc as plsc

N = 10000
D = 128
PD = 144
NC, NS, L = 2, 16, 16
NW = NC * NS
E_RAW = 320000
E_REAL = E_RAW + N
CHUNK = 128
NCHUNK = 81
T_TILE = NCHUNK * CHUNK
E_PAD = T_TILE * NW
NP = 10240
RPT = NP // NS
BR = 1000
BRP = 1280


def _pre_body(x_ref, w_ref, asw_ref, adw_ref, hp_ref, as_ref, ad_ref):
    h = jnp.dot(x_ref[...], w_ref[...], preferred_element_type=jnp.float32)
    a_s = (h * asw_ref[...]).sum(axis=1)
    a_d = (h * adw_ref[...]).sum(axis=1)
    hp_ref[:, :D] = h
    col = lax.broadcasted_iota(jnp.int32, (BRP, PD - D), 1)
    hp_ref[:, D:] = jnp.where(col == 0, 1.0,
                              jnp.where(col == 1, a_s[:, None], 0.0))
    i = pl.program_id(0)
    as_ref[pl.ds(pl.multiple_of(i * BRP, 128), BRP)] = a_s
    ad_ref[pl.ds(pl.multiple_of(i * BRP, 128), BRP)] = a_d


def _post_body(part_ref, bias_ref, o_ref):
    p = part_ref[...]
    srow = p[0] + p[1]
    num = srow[:, :D]
    den = srow[:, D:D + 1]
    o_ref[...] = jnp.tanh(num / (den + 1e-16) + bias_ref[...])


def _sc_body(hp_hbm, edges_hbm, adst_hbm, g_hbm, out_hbm,
             gv, ed0, ed1, ad0, ad1, ea0, ea1, rows0, rows1, acc,
             sg0, sg1, sa0, sa1, ss0, ss1):
    c = lax.axis_index("c")
    s = lax.axis_index("s")
    wid = c * NS + s
    pltpu.sync_copy(g_hbm, gv)

    def _zrow(i, carry):
        for j in range(PD // L):
            rows0[i, pl.ds(j * L, L)] = jnp.zeros((L,), jnp.float32)
        return carry
    lax.fori_loop(0, CHUNK, _zrow, 0)
    for b in range(RPT // CHUNK):
        pltpu.sync_copy(rows0,
                        acc.at[pl.ds(s * RPT + b * CHUNK, CHUNK), :])
    plsc.subcore_barrier()

    gvec = gv[...]
    lane = lax.iota(jnp.int32, L)

    def _edge_dma(k, ed):
        pltpu.sync_copy(edges_hbm.at[wid, :, pl.ds(k * CHUNK, CHUNK)], ed)

    def _compute(k, ed, ad_v, ea_v, rows_v):
        base = wid * T_TILE + k * CHUNK
        for g8 in range(CHUNK // L):
            ridx = g8 * L + lane
            a_s = plsc.load_gather(rows_v, [ridx,
                                            jnp.full((L,), D + 1, jnp.int32)])
            al = a_s + ad_v[pl.ds(g8 * L, L)]
            al = jnp.where(al > 0, al, 0.2 * al)
            ea = jnp.exp(al - gvec)
            eid = base + ridx
            ea = jnp.where(eid < E_REAL, ea, 0.0)
            ea_v[pl.ds(g8 * L, L)] = ea

        @plsc.parallel_loop(0, CHUNK, 1, unroll=8)
        def _scale(e):
            eb = plsc.load_gather(ea_v, [jnp.zeros((L,), jnp.int32) + e])
            for j in range(PD // L):
                rows_v[e, pl.ds(j * L, L)] = rows_v[e, pl.ds(j * L, L)] * eb

    def _half(k, ed_b, ad_b, ea_b, rows_b, sg_b, sa_b, ss_b,
              ed_n, ad_n, rows_n, sg_n, sa_n, ss_n, first, last):
        pltpu.make_async_copy(hp_hbm.at[ed_b.at[0]], rows_b, sg_b).wait()
        pltpu.make_async_copy(adst_hbm.at[ed_b.at[1]], ad_b, sa_b).wait()
        _compute(k, ed_b, ad_b, ea_b, rows_b)
        pltpu.async_copy(rows_b, acc.at[ed_b.at[1]], ss_b, add=True)
        if not first:
            pltpu.make_async_copy(rows_n, acc.at[ed_n.at[1]], ss_n).wait()
        if not last:
            _edge_dma(k + 1, ed_n)
            pltpu.async_copy(hp_hbm.at[ed_n.at[0]], rows_n, sg_n)
            pltpu.async_copy(adst_hbm.at[ed_n.at[1]], ad_n, sa_n)

    _edge_dma(0, ed0)
    pltpu.async_copy(hp_hbm.at[ed0.at[0]], rows0, sg0)
    pltpu.async_copy(adst_hbm.at[ed0.at[1]], ad0, sa0)
    _half(0, ed0, ad0, ea0, rows0, sg0, sa0, ss0,
          ed1, ad1, rows1, sg1, sa1, ss1, True, False)

    NPAIR = (NCHUNK - 1) // 2

    def _pair(kk, carry):
        @pl.when(kk < NPAIR - 1)
        def _():
            _half(2 * kk + 1, ed1, ad1, ea1, rows1, sg1, sa1, ss1,
                  ed0, ad0, rows0, sg0, sa0, ss0, False, False)
            _half(2 * kk + 2, ed0, ad0, ea0, rows0, sg0, sa0, ss0,
                  ed1, ad1, rows1, sg1, sa1, ss1, False, False)

        @pl.when(kk == NPAIR - 1)
        def _():
            _half(2 * kk + 1, ed1, ad1, ea1, rows1, sg1, sa1, ss1,
                  ed0, ad0, rows0, sg0, sa0, ss0, False, False)
            _half(2 * kk + 2, ed0, ad0, ea0, rows0, sg0, sa0, ss0,
                  ed1, ad1, rows1, sg1, sa1, ss1, False, True)
        return carry
    lax.fori_loop(0, NPAIR, _pair, 0)
    pltpu.make_async_copy(rows0, acc.at[ed0.at[1]], ss0).wait()

    plsc.subcore_barrier()
    for b in range(RPT // CHUNK):
        r0 = s * RPT + b * CHUNK
        pltpu.sync_copy(acc.at[pl.ds(r0, CHUNK), :],
                        out_hbm.at[c, pl.ds(r0, CHUNK), :])


def kernel(x, edge_index, W, att_src, att_dst, bias):
    xp = jnp.concatenate([x, jnp.zeros((NP - N, D), jnp.float32)], axis=0)
    hp, a_src, a_dst = pl.pallas_call(
        _pre_body,
        grid=(NP // BRP,),
        in_specs=[
            pl.BlockSpec((BRP, D), lambda i: (i, 0)),
            pl.BlockSpec((D, D), lambda i: (0, 0)),
            pl.BlockSpec((1, D), lambda i: (0, 0)),
            pl.BlockSpec((1, D), lambda i: (0, 0)),
        ],
        out_specs=[
            pl.BlockSpec((BRP, PD), lambda i: (i, 0)),
            pl.BlockSpec((NP,), lambda i: (0,)),
            pl.BlockSpec((NP,), lambda i: (0,)),
        ],
        out_shape=[
            jax.ShapeDtypeStruct((NP, PD), jnp.float32),
            jax.ShapeDtypeStruct((NP,), jnp.float32),
            jax.ShapeDtypeStruct((NP,), jnp.float32),
        ],
    )(xp, W, att_src, att_dst)

    bound = jnp.max(a_src[:N]) + jnp.max(a_dst[:N])
    g = jnp.where(bound > 0, bound, 0.2 * bound)
    g16 = jnp.full((L,), g, jnp.float32)

    loop = jnp.arange(N, dtype=jnp.int32)
    padz = jnp.zeros((E_PAD - E_REAL,), jnp.int32)
    srcp = jnp.concatenate([edge_index[0], loop, padz]).reshape(NW, T_TILE)
    dstp = jnp.concatenate([edge_index[1], loop, padz]).reshape(NW, T_TILE)
    edges = jnp.stack([srcp, dstp], axis=1)

    mesh = plsc.VectorSubcoreMesh(core_axis_name="c", subcore_axis_name="s")
    part = pl.kernel(
        _sc_body,
        mesh=mesh,
        compiler_params=pltpu.CompilerParams(
            needs_layout_passes=False, use_tc_tiling_on_sc=False),
        out_type=jax.ShapeDtypeStruct((NC, NP, PD), jnp.float32),
        scratch_types=[
            pltpu.VMEM((L,), jnp.float32),
            pltpu.VMEM((2, CHUNK), jnp.int32),
            pltpu.VMEM((2, CHUNK), jnp.int32),
            pltpu.VMEM((CHUNK,), jnp.float32),
            pltpu.VMEM((CHUNK,), jnp.float32),
            pltpu.VMEM((CHUNK,), jnp.float32),
            pltpu.VMEM((CHUNK,), jnp.float32),
            pltpu.VMEM((CHUNK, PD), jnp.float32),
            pltpu.VMEM((CHUNK, PD), jnp.float32),
            pltpu.VMEM_SHARED((NP, PD), jnp.float32),
            pltpu.SemaphoreType.DMA,
            pltpu.SemaphoreType.DMA,
            pltpu.SemaphoreType.DMA,
            pltpu.SemaphoreType.DMA,
            pltpu.SemaphoreType.DMA,
            pltpu.SemaphoreType.DMA,
        ],
    )(hp, edges, a_dst, g16)

    out = pl.pallas_call(
        _post_body,
        grid=(N // BR,),
        in_specs=[
            pl.BlockSpec((NC, BR, PD), lambda i: (0, i, 0)),
            pl.BlockSpec((D,), lambda i: (0,)),
        ],
        out_specs=pl.BlockSpec((BR, D), lambda i: (i, 0)),
        out_shape=jax.ShapeDtypeStruct((N, D), jnp.float32),
    )(part, bias)
    return out


# speedup vs baseline: 33.6244x; 1.1813x over previous
import functools

import jax
import jax.numpy as jnp
from jax import lax
from jax.experimental import pallas as pl
from jax.experimental.pallas import tpu as pltpu
from jax.experimental.pallas import tpu_sc as plsc

N = 10000
D = 128
PD = 144
NC, NS, L = 2, 16, 16
NW = NC * NS
E_RAW = 320000
E_REAL = E_RAW + N
CHUNK = 128
NCHUNK = 81
T_TILE = NCHUNK * CHUNK
E_PAD = T_TILE * NW
NP = 10240
RPT = NP // NS
BR = 1000
BRP = 1280


def _pre_body(x_ref, w_ref, asw_ref, adw_ref, hp_ref, as_ref, ad_ref):
    h = jnp.dot(x_ref[...], w_ref[...], preferred_element_type=jnp.float32)
    a_s = (h * asw_ref[...]).sum(axis=1)
    a_d = (h * adw_ref[...]).sum(axis=1)
    hp_ref[:, :D] = h
    col = lax.broadcasted_iota(jnp.int32, (BRP, PD - D), 1)
    hp_ref[:, D:] = jnp.where(col == 0, 1.0,
                              jnp.where(col == 1, a_s[:, None], 0.0))
    i = pl.program_id(0)
    as_ref[pl.ds(pl.multiple_of(i * BRP, 128), BRP)] = a_s
    ad_ref[pl.ds(pl.multiple_of(i * BRP, 128), BRP)] = a_d


def _post_body(part_ref, bias_ref, o_ref):
    p = part_ref[...]
    srow = p[0] + p[1]
    num = srow[:, :D]
    den = srow[:, D:D + 1]
    o_ref[...] = jnp.tanh(num / (den + 1e-16) + bias_ref[...])


def _sc_body(hp_hbm, edges_hbm, adst_hbm, g_hbm, out_hbm,
             gv, ed0, ed1, ad0, ad1, ea0, ea1, rows0, rows1, acc,
             sg0, sg1, sa0, sa1, ss0, ss1):
    c = lax.axis_index("c")
    s = lax.axis_index("s")
    wid = c * NS + s
    pltpu.sync_copy(g_hbm, gv)

    def _zrow(i, carry):
        for j in range(PD // L):
            rows0[i, pl.ds(j * L, L)] = jnp.zeros((L,), jnp.float32)
        return carry
    lax.fori_loop(0, CHUNK, _zrow, 0)
    for b in range(RPT // CHUNK):
        pltpu.sync_copy(rows0,
                        acc.at[pl.ds(s * RPT + b * CHUNK, CHUNK), :])
    plsc.subcore_barrier()

    gvec = gv[...]
    lane = lax.iota(jnp.int32, L)

    def _edge_dma(k, ed):
        pltpu.sync_copy(edges_hbm.at[wid, :, pl.ds(k * CHUNK, CHUNK)], ed)

    def _compute(k, ed, ad_v, ea_v, rows_v):
        base = wid * T_TILE + k * CHUNK
        for g8 in range(CHUNK // L):
            ridx = g8 * L + lane
            a_s = plsc.load_gather(rows_v, [ridx,
                                            jnp.full((L,), D + 1, jnp.int32)])
            al = a_s + ad_v[pl.ds(g8 * L, L)]
            al = jnp.where(al > 0, al, 0.2 * al)
            ea = jnp.exp(al - gvec)
            eid = base + ridx
            ea = jnp.where(eid < E_REAL, ea, 0.0)
            ea_v[pl.ds(g8 * L, L)] = ea

        @plsc.parallel_loop(0, CHUNK, 1, unroll=8)
        def _scale(e):
            eb = plsc.load_gather(ea_v, [jnp.zeros((L,), jnp.int32) + e])
            for j in range(PD // L):
                rows_v[e, pl.ds(j * L, L)] = rows_v[e, pl.ds(j * L, L)] * eb

    def _half(k, ed_b, ad_b, ea_b, rows_b, sg_b, sa_b, ss_b,
              ed_n, ad_n, rows_n, sg_n, sa_n, ss_n, first, last):
        if not first:
            pltpu.make_async_copy(rows_n, acc.at[ed_n.at[1]], ss_n).wait()
        if not last:
            _edge_dma(k + 1, ed_n)
            pltpu.async_copy(hp_hbm.at[ed_n.at[0]], rows_n, sg_n)
            pltpu.async_copy(adst_hbm.at[ed_n.at[1]], ad_n, sa_n)
        pltpu.make_async_copy(hp_hbm.at[ed_b.at[0]], rows_b, sg_b).wait()
        pltpu.make_async_copy(adst_hbm.at[ed_b.at[1]], ad_b, sa_b).wait()
        _compute(k, ed_b, ad_b, ea_b, rows_b)
        pltpu.async_copy(rows_b, acc.at[ed_b.at[1]], ss_b, add=True)

    _edge_dma(0, ed0)
    pltpu.async_copy(hp_hbm.at[ed0.at[0]], rows0, sg0)
    pltpu.async_copy(adst_hbm.at[ed0.at[1]], ad0, sa0)
    _half(0, ed0, ad0, ea0, rows0, sg0, sa0, ss0,
          ed1, ad1, rows1, sg1, sa1, ss1, True, False)

    NPAIR = (NCHUNK - 1) // 2

    def _pair(kk, carry):
        @pl.when(kk < NPAIR - 1)
        def _():
            _half(2 * kk + 1, ed1, ad1, ea1, rows1, sg1, sa1, ss1,
                  ed0, ad0, rows0, sg0, sa0, ss0, False, False)
            _half(2 * kk + 2, ed0, ad0, ea0, rows0, sg0, sa0, ss0,
                  ed1, ad1, rows1, sg1, sa1, ss1, False, False)

        @pl.when(kk == NPAIR - 1)
        def _():
            _half(2 * kk + 1, ed1, ad1, ea1, rows1, sg1, sa1, ss1,
                  ed0, ad0, rows0, sg0, sa0, ss0, False, False)
            _half(2 * kk + 2, ed0, ad0, ea0, rows0, sg0, sa0, ss0,
                  ed1, ad1, rows1, sg1, sa1, ss1, False, True)
        return carry
    lax.fori_loop(0, NPAIR, _pair, 0)
    pltpu.make_async_copy(rows0, acc.at[ed0.at[1]], ss0).wait()

    plsc.subcore_barrier()
    for b in range(RPT // CHUNK):
        r0 = s * RPT + b * CHUNK
        pltpu.sync_copy(acc.at[pl.ds(r0, CHUNK), :],
                        out_hbm.at[c, pl.ds(r0, CHUNK), :])


def kernel(x, edge_index, W, att_src, att_dst, bias):
    xp = jnp.concatenate([x, jnp.zeros((NP - N, D), jnp.float32)], axis=0)
    hp, a_src, a_dst = pl.pallas_call(
        _pre_body,
        grid=(NP // BRP,),
        in_specs=[
            pl.BlockSpec((BRP, D), lambda i: (i, 0)),
            pl.BlockSpec((D, D), lambda i: (0, 0)),
            pl.BlockSpec((1, D), lambda i: (0, 0)),
            pl.BlockSpec((1, D), lambda i: (0, 0)),
        ],
        out_specs=[
            pl.BlockSpec((BRP, PD), lambda i: (i, 0)),
            pl.BlockSpec((NP,), lambda i: (0,)),
            pl.BlockSpec((NP,), lambda i: (0,)),
        ],
        out_shape=[
            jax.ShapeDtypeStruct((NP, PD), jnp.float32),
            jax.ShapeDtypeStruct((NP,), jnp.float32),
            jax.ShapeDtypeStruct((NP,), jnp.float32),
        ],
    )(xp, W, att_src, att_dst)

    bound = jnp.max(a_src[:N]) + jnp.max(a_dst[:N])
    g = jnp.where(bound > 0, bound, 0.2 * bound)
    g16 = jnp.full((L,), g, jnp.float32)

    loop = jnp.arange(N, dtype=jnp.int32)
    padz = jnp.zeros((E_PAD - E_REAL,), jnp.int32)
    srcp = jnp.concatenate([edge_index[0], loop, padz]).reshape(NW, T_TILE)
    dstp = jnp.concatenate([edge_index[1], loop, padz]).reshape(NW, T_TILE)
    edges = jnp.stack([srcp, dstp], axis=1)

    mesh = plsc.VectorSubcoreMesh(core_axis_name="c", subcore_axis_name="s")
    part = pl.kernel(
        _sc_body,
        mesh=mesh,
        compiler_params=pltpu.CompilerParams(
            needs_layout_passes=False, use_tc_tiling_on_sc=False),
        out_type=jax.ShapeDtypeStruct((NC, NP, PD), jnp.float32),
        scratch_types=[
            pltpu.VMEM((L,), jnp.float32),
            pltpu.VMEM((2, CHUNK), jnp.int32),
            pltpu.VMEM((2, CHUNK), jnp.int32),
            pltpu.VMEM((CHUNK,), jnp.float32),
            pltpu.VMEM((CHUNK,), jnp.float32),
            pltpu.VMEM((CHUNK,), jnp.float32),
            pltpu.VMEM((CHUNK,), jnp.float32),
            pltpu.VMEM((CHUNK, PD), jnp.float32),
            pltpu.VMEM((CHUNK, PD), jnp.float32),
            pltpu.VMEM_SHARED((NP, PD), jnp.float32),
            pltpu.SemaphoreType.DMA,
            pltpu.SemaphoreType.DMA,
            pltpu.SemaphoreType.DMA,
            pltpu.SemaphoreType.DMA,
            pltpu.SemaphoreType.DMA,
            pltpu.SemaphoreType.DMA,
        ],
    )(hp, edges, a_dst, g16)

    out = pl.pallas_call(
        _post_body,
        grid=(N // BR,),
        in_specs=[
            pl.BlockSpec((NC, BR, PD), lambda i: (0, i, 0)),
            pl.BlockSpec((D,), lambda i: (0,)),
        ],
        out_specs=pl.BlockSpec((BR, D), lambda i: (i, 0)),
        out_shape=jax.ShapeDtypeStruct((N, D), jnp.float32),
    )(part, bias)
    return out
